# speedup vs baseline: 11.4538x; 11.4538x over previous
import numpy as np
import jax
import jax.numpy as jnp
from jax.experimental import pallas as pl
from jax.experimental.pallas import tpu as pltpu

D = 128
L = 20
NVALS = 8
B = 512


def _pe_table() -> jax.Array:
    half = D // 2
    i = np.arange(half, dtype=np.float64)
    div = np.exp(-(np.log(10000.0)) * (2.0 * i) / D)
    p = np.arange(NVALS, dtype=np.float64)[:, None]
    ang = p * div[None, :]
    tab = np.concatenate([np.sin(ang), np.cos(ang)], axis=-1)
    tab[0] = 0.0
    return jnp.asarray(tab, dtype=jnp.float32)


def _tc_body(ids_ref, pos_ref, table_ref, petab_ref, out_ref):
    i = pl.program_id(0)
    ids = ids_ref[pl.ds(i * B, B)]
    vocab = table_ref.shape[0]
    onehot = (ids[:, None] == jax.lax.broadcasted_iota(
        jnp.int32, (1, vocab), 1)).astype(jnp.float32)
    node_vec = jnp.dot(onehot, table_ref[...],
                       preferred_element_type=jnp.float32)

    pos = pos_ref[...]
    acc = jnp.zeros((B, D), jnp.float32)
    for p in range(1, NVALS):
        cnt = jnp.sum((pos == p).astype(jnp.float32), axis=1,
                      keepdims=True)
        acc = acc + cnt * petab_ref[p, :][None, :]
    norm = jnp.sqrt(jnp.sum(acc * acc, axis=1, keepdims=True))
    acc = acc / (norm + 1e-8)
    out_ref[...] = node_vec + acc


def kernel(node_ids, positions, table):
    n = node_ids.shape[0]
    petab = _pe_table()
    grid = (n // B,)
    return pl.pallas_call(
        _tc_body,
        grid=grid,
        in_specs=[
            pl.BlockSpec((n,), lambda i: (0,)),
            pl.BlockSpec((B, L), lambda i: (i, 0)),
            pl.BlockSpec(table.shape, lambda i: (0, 0)),
            pl.BlockSpec((NVALS, D), lambda i: (0, 0)),
        ],
        out_specs=pl.BlockSpec((B, D), lambda i: (i, 0)),
        out_shape=jax.ShapeDtypeStruct((n, D), jnp.float32),
    )(node_ids, positions, table, petab)
